# Initial kernel scaffold; baseline (speedup 1.0000x reference)
import jax
import jax.numpy as jnp
from jax.experimental import pallas as pl


def kernel(batch_box_preds, batch_cls_preds, batch_size):
    raise NotImplementedError("write your pallas kernel here")


# speedup vs baseline: 15.6493x; 15.6493x over previous
import jax
import jax.numpy as jnp
from jax.experimental import pallas as pl
from jax.experimental.pallas import tpu as pltpu

_N = 5000
_NPAD = 5120
_M = 2048
_K = 512
_TH = 0.7
_NEG = -1e30
_CH = 256


def _split3(x):
    b = jax.lax.bitcast_convert_type(x, jnp.uint32)
    hi = jax.lax.bitcast_convert_type(b & jnp.uint32(0xFFFF0000), jnp.float32)
    r = x - hi
    rb = jax.lax.bitcast_convert_type(r, jnp.uint32)
    mid = jax.lax.bitcast_convert_type(rb & jnp.uint32(0xFFFF0000), jnp.float32)
    lo = r - mid
    return (hi.astype(jnp.bfloat16), mid.astype(jnp.bfloat16),
            lo.astype(jnp.bfloat16))


def _dot(a, b):
    return jax.lax.dot_general(a, b, (((1,), (0,)), ((), ())),
                               preferred_element_type=jnp.float32)


def _col(mat, j):
    lane = jax.lax.broadcasted_iota(jnp.int32, mat.shape, 1)
    return jnp.sum(jnp.where(lane == j, mat, 0.0), axis=1, keepdims=True)


def _nms_body(cls_rows_ref, cls_cols_ref, boxg_ref, box_rows_ref, out_ref,
              o_ref, r_ref, w_ref):
    f32 = jnp.float32
    bf16 = jnp.bfloat16

    cls_rows = cls_rows_ref[0]
    cls_cols = cls_cols_ref[0]

    s_row = jnp.max(cls_rows, axis=0, keepdims=True)
    s_col = jnp.max(cls_cols, axis=1, keepdims=True)
    lane = jax.lax.broadcasted_iota(jnp.int32, (_NPAD, 128), 1)
    lab_col = jnp.min(jnp.where(cls_cols == s_col, lane, 128), axis=1,
                      keepdims=True)

    i_row = jax.lax.broadcasted_iota(jnp.int32, (1, _NPAD), 1)
    i_col = jax.lax.broadcasted_iota(jnp.int32, (_NPAD, 1), 0)
    rank_row = jnp.zeros((1, _NPAD), f32)
    rank_col = jnp.zeros((_NPAD, 1), f32)
    for c in range(_NPAD // _CH):
        sj = jax.lax.slice(s_col, (c * _CH, 0), ((c + 1) * _CH, 1))
        jidx = jax.lax.broadcasted_iota(jnp.int32, (_CH, 1), 0) + c * _CH
        beats = (sj > s_row) | ((sj == s_row) & (jidx < i_row))
        rank_row = rank_row + jnp.sum(beats.astype(f32), axis=0, keepdims=True)
        sj_r = jax.lax.slice(s_row, (0, c * _CH), (1, (c + 1) * _CH))
        jidx_r = jax.lax.broadcasted_iota(jnp.int32, (1, _CH), 1) + c * _CH
        beats2 = (sj_r > s_col) | ((sj_r == s_col) & (jidx_r < i_col))
        rank_col = rank_col + jnp.sum(beats2.astype(f32), axis=1, keepdims=True)

    G = boxg_ref[0]
    G = jnp.where(lane == 7, s_col, G)
    G = jnp.where(lane == 8, lab_col.astype(f32), G)
    gh, gm, gl = _split3(G)
    for c in range(_M // _CH):
        rr = (jax.lax.broadcasted_iota(jnp.int32, (_CH, 1), 0)
              + c * _CH).astype(f32)
        P = (rank_row == rr).astype(bf16)
        r_ref[c * _CH:(c + 1) * _CH, :] = (_dot(P, gh) + _dot(P, gm)
                                           + _dot(P, gl))

    bx = box_rows_ref[0]
    bh, bm, bl = _split3(bx)
    for c in range(_M // _CH):
        rr = (jax.lax.broadcasted_iota(jnp.int32, (1, _CH), 1)
              + c * _CH).astype(f32)
        PT = (rank_col == rr).astype(bf16)
        w_ref[:, c * _CH:(c + 1) * _CH] = (_dot(bh, PT) + _dot(bm, PT)
                                           + _dot(bl, PT))

    Rv = r_ref[:, :]
    x_c = _col(Rv, 0)
    y_c = _col(Rv, 1)
    dx_c = _col(Rv, 3)
    dy_c = _col(Rv, 4)
    x1c = x_c - dx_c * 0.5
    x2c = x_c + dx_c * 0.5
    y1c = y_c - dy_c * 0.5
    y2c = y_c + dy_c * 0.5
    area_c = (x2c - x1c) * (y2c - y1c)

    Wv = w_ref[:, :]
    x_r = Wv[0:1, :]
    y_r = Wv[1:2, :]
    dx_r = Wv[3:4, :]
    dy_r = Wv[4:5, :]
    x1r = x_r - dx_r * 0.5
    x2r = x_r + dx_r * 0.5
    y1r = y_r - dy_r * 0.5
    y2r = y_r + dy_r * 0.5
    area_r = (x2r - x1r) * (y2r - y1r)

    jj = jax.lax.broadcasted_iota(jnp.int32, (1, _M), 1)
    for c in range(_M // _CH):
        sl = lambda v: jax.lax.slice(v, (c * _CH, 0), ((c + 1) * _CH, 1))
        xx1 = jnp.maximum(sl(x1c), x1r)
        xx2 = jnp.minimum(sl(x2c), x2r)
        yy1 = jnp.maximum(sl(y1c), y1r)
        yy2 = jnp.minimum(sl(y2c), y2r)
        inter = jnp.clip(xx2 - xx1, 0.0) * jnp.clip(yy2 - yy1, 0.0)
        iou = inter / (sl(area_c) + area_r - inter + 1e-6)
        ii = jax.lax.broadcasted_iota(jnp.int32, (_CH, 1), 0) + c * _CH
        o_ref[c * _CH:(c + 1) * _CH, :] = ((iou > _TH) & (jj > ii)).astype(bf16)

    Ov = o_ref[:, :]

    def cond(carry):
        return carry[1]

    def body(carry):
        a, _ = carry
        hits = _dot(a.astype(bf16), Ov)
        a_new = (hits == 0.0).astype(f32)
        return a_new, jnp.any(a_new != a)

    a0 = jnp.ones((1, _M), f32)
    keep, _ = jax.lax.while_loop(cond, body, (a0, jnp.array(True)))

    kb = keep.astype(bf16)
    jcol = jax.lax.broadcasted_iota(jnp.int32, (_M, 1), 0)
    pieces = []
    for c in range(_M // _CH):
        irow = jax.lax.broadcasted_iota(jnp.int32, (1, _CH), 1) + c * _CH
        U = (jcol <= irow).astype(bf16)
        pieces.append(_dot(kb, U))
    rank2 = jnp.concatenate(pieces, axis=1) - 1.0
    validm = (keep > 0.0) & (rank2 < float(_K))
    pos = jnp.where(validm, rank2, float(_K))
    qr = jax.lax.broadcasted_iota(jnp.int32, (_K, 1), 0).astype(f32)
    Q = (pos == qr).astype(bf16)
    rh, rm, rl = _split3(Rv)
    out_ref[0] = _dot(Q, rh) + _dot(Q, rm) + _dot(Q, rl)


def kernel(batch_box_preds, batch_cls_preds, batch_size):
    f32 = jnp.float32
    B, N, C = batch_cls_preds.shape
    boxes = batch_box_preds.astype(f32)
    cls = batch_cls_preds.astype(f32)

    boxg = jnp.zeros((B, _NPAD, 128), f32).at[:, :N, :7].set(boxes)
    box_rows = jnp.zeros((B, 8, _NPAD), f32).at[:, :7, :N].set(
        boxes.transpose(0, 2, 1))
    clsc = jnp.full((B, _NPAD, 128), _NEG, f32).at[:, :N, :C].set(cls)
    clsr = jnp.full((B, 8, _NPAD), _NEG, f32).at[:, :C, :N].set(
        cls.transpose(0, 2, 1))

    out = pl.pallas_call(
        _nms_body,
        grid=(B,),
        in_specs=[
            pl.BlockSpec((1, 8, _NPAD), lambda b: (b, 0, 0)),
            pl.BlockSpec((1, _NPAD, 128), lambda b: (b, 0, 0)),
            pl.BlockSpec((1, _NPAD, 128), lambda b: (b, 0, 0)),
            pl.BlockSpec((1, 8, _NPAD), lambda b: (b, 0, 0)),
        ],
        out_specs=pl.BlockSpec((1, _K, 128), lambda b: (b, 0, 0)),
        out_shape=jax.ShapeDtypeStruct((B, _K, 128), f32),
        scratch_shapes=[
            pltpu.VMEM((_M, _M), jnp.bfloat16),
            pltpu.VMEM((_M, 128), f32),
            pltpu.VMEM((8, _M), f32),
        ],
        compiler_params=pltpu.CompilerParams(
            dimension_semantics=("arbitrary",),
            vmem_limit_bytes=120 * 1024 * 1024,
        ),
    )(clsr, clsc, boxg, box_rows)

    rois = out[:, :, :7]
    roi_scores = out[:, :, 7]
    roi_labels = out[:, :, 8].astype(jnp.int32) + 1
    return rois, roi_scores, roi_labels
